# Initial kernel scaffold; baseline (speedup 1.0000x reference)
import jax
import jax.numpy as jnp
from jax.experimental import pallas as pl


def kernel(x, table, pos_emb):
    raise NotImplementedError("write your pallas kernel here")


# speedup vs baseline: 1.0760x; 1.0760x over previous
import functools

import jax
import jax.numpy as jnp
from jax import lax
from jax.experimental import pallas as pl
from jax.experimental.pallas import tpu as pltpu
from jax.experimental.pallas import tpu_sc as plsc

DIM = 128
CTX = 2048
B = 4
TOK = B * CTX
NC, NS, LANES = 2, 16, 16
NW = NC * NS
TPW = TOK // NW
GCHUNK = 128
NG = TPW // GCHUNK


@functools.partial(
    pl.kernel,
    out_type=jax.ShapeDtypeStruct((TOK, DIM), jnp.float32),
    mesh=plsc.VectorSubcoreMesh(core_axis_name="c", subcore_axis_name="s"),
    scratch_types=[
        pltpu.VMEM((NG, GCHUNK), jnp.int32),
        pltpu.VMEM((TPW, DIM), jnp.float32),
        pltpu.VMEM((TPW, DIM), jnp.float32),
        pltpu.SemaphoreType.DMA,
    ],
)
def _embed_sc(x_hbm, table_hbm, pos_hbm, out_hbm, idx_v, rows_v, pos_v, sem):
    wid = lax.axis_index("s") * NC + lax.axis_index("c")
    base = wid * TPW

    pltpu.sync_copy(x_hbm.at[pl.ds(wid * NG, NG)], idx_v)

    copies = [
        pltpu.async_copy(
            table_hbm.at[idx_v.at[j]],
            rows_v.at[pl.ds(j * GCHUNK, GCHUNK)],
            sem,
        )
        for j in range(NG)
    ]

    pos_base = lax.rem(base, CTX)
    pltpu.sync_copy(pos_hbm.at[pl.ds(pos_base, TPW)], pos_v)

    for cp in copies:
        cp.wait()

    def row_body(i, carry):
        for c in range(DIM // LANES):
            s = pl.ds(c * LANES, LANES)
            rows_v[i, s] = rows_v[i, s] + pos_v[i, s]
        return carry

    lax.fori_loop(0, TPW, row_body, 0)

    pltpu.sync_copy(rows_v, out_hbm.at[pl.ds(base, TPW)])


def kernel(x, table, pos_emb):
    x_rows = x.reshape(NW * NG, GCHUNK).astype(jnp.int32)
    out = _embed_sc(x_rows, table, pos_emb)
    return out.reshape(B, CTX, DIM)


# speedup vs baseline: 1.1458x; 1.0649x over previous
import functools

import jax
import jax.numpy as jnp
from jax import lax
from jax.experimental import pallas as pl
from jax.experimental.pallas import tpu as pltpu
from jax.experimental.pallas import tpu_sc as plsc

DIM = 128
CTX = 2048
B = 4
TOK = B * CTX
NC, NS, LANES = 2, 16, 16
NW = NC * NS
NPOS = CTX // NW


@functools.partial(
    pl.kernel,
    out_type=jax.ShapeDtypeStruct((TOK, DIM), jnp.float32),
    mesh=plsc.VectorSubcoreMesh(core_axis_name="c", subcore_axis_name="s"),
    scratch_types=[
        pltpu.VMEM((B, NPOS), jnp.int32),
        pltpu.VMEM((NPOS, DIM), jnp.float32),
        pltpu.VMEM((B * NPOS, DIM), jnp.float32),
        pltpu.SemaphoreType.DMA,
        pltpu.SemaphoreType.DMA,
        pltpu.SemaphoreType.DMA,
        pltpu.SemaphoreType.DMA,
        pltpu.SemaphoreType.DMA,
        pltpu.SemaphoreType.DMA,
    ],
)
def _embed_sc(x_hbm, table_hbm, pos_hbm, out_hbm,
              idx_v, pos_v, rows_v, g0, g1, g2, g3, psem, ssem):
    wid = lax.axis_index("s") * NC + lax.axis_index("c")
    pbase = wid * NPOS

    gsems = (g0, g1, g2, g3)
    idx_cps = [
        pltpu.async_copy(
            x_hbm.at[b, pl.ds(pbase, NPOS)], idx_v.at[b], gsems[b]
        )
        for b in range(B)
    ]
    gathers = []
    for b in range(B):
        idx_cps[b].wait()
        gathers.append(
            pltpu.async_copy(
                table_hbm.at[idx_v.at[b]],
                rows_v.at[pl.ds(b * NPOS, NPOS)],
                gsems[b],
            )
        )
    pcp = pltpu.async_copy(pos_hbm.at[pl.ds(pbase, NPOS)], pos_v, psem)
    pcp.wait()

    stores = []
    for b in range(B):
        gathers[b].wait()
        bofs = b * NPOS

        def row_body(i, carry):
            for c in range(DIM // LANES):
                s = pl.ds(c * LANES, LANES)
                plsc.addupdate(rows_v.at[bofs + i, s], pos_v[i, s])
            return carry

        lax.fori_loop(0, NPOS, row_body, 0)
        stores.append(
            pltpu.async_copy(
                rows_v.at[pl.ds(bofs, NPOS)],
                out_hbm.at[pl.ds(b * CTX + pbase, NPOS)],
                ssem,
            )
        )

    for cp in stores:
        cp.wait()


def kernel(x, table, pos_emb):
    out = _embed_sc(x.astype(jnp.int32), table, pos_emb)
    return out.reshape(B, CTX, DIM)
